# Initial kernel scaffold; baseline (speedup 1.0000x reference)
import jax
import jax.numpy as jnp
from jax.experimental import pallas as pl


def kernel(r_idx, r_weight, kv):
    raise NotImplementedError("write your pallas kernel here")


# speedup vs baseline: 9.3324x; 9.3324x over previous
import functools

import jax
import jax.numpy as jnp
from jax import lax
from jax.experimental import pallas as pl
from jax.experimental.pallas import tpu as pltpu
from jax.experimental.pallas import tpu_sc as plsc

N, P2, W2, TOPK, CKV = 8, 49, 64, 4, 192
R = N * P2 * W2
B = N * P2 * W2 * TOPK
NC, NS, L = 2, 16, 16
NW = NC * NS
BPW = B // NW
CH = 112
NCH = BPW // CH
VPC = CH // L

_mesh = plsc.VectorSubcoreMesh(core_axis_name="c", subcore_axis_name="s")


@functools.partial(
    pl.kernel,
    mesh=_mesh,
    compiler_params=pltpu.CompilerParams(use_tc_tiling_on_sc=False),
    out_type=jax.ShapeDtypeStruct((B, CKV), jnp.float32),
    scratch_types=[
        pltpu.VMEM((BPW,), jnp.int32),
        pltpu.VMEM((NCH, CH), jnp.int32),
        pltpu.VMEM((CH, CKV), jnp.float32),
        pltpu.VMEM((CH, CKV), jnp.float32),
        pltpu.SemaphoreType.DMA,
        pltpu.SemaphoreType.DMA,
    ],
)
def _sc_gather(idx_hbm, kv_hbm, out_hbm, idx_v, gidx_v, buf0, buf1, sem0, sem1):
    wid = lax.axis_index("s") * NC + lax.axis_index("c")
    rbase = wid * BPW

    pltpu.sync_copy(idx_hbm.at[pl.ds(rbase, BPW)], idx_v)

    lane = lax.iota(jnp.int32, 16)

    def ibody(c, _):
        for j in range(VPC):
            off = c * CH + j * L
            rid = rbase + off + lane
            gidx_v[c, pl.ds(j * L, L)] = idx_v[pl.ds(off, L)] + ((rid >> 8) << 6)
        return 0

    lax.fori_loop(0, NCH, ibody, 0)

    bufs = (buf0, buf1)
    sems = (sem0, sem1)

    def gather(c, slot):
        return pltpu.async_copy(kv_hbm.at[gidx_v.at[c]], bufs[slot], sems[slot])

    handles = [gather(0, 0), None]
    for c in range(NCH):
        slot = c % 2
        if c + 1 < NCH:
            handles[1 - slot] = gather(c + 1, 1 - slot)
        handles[slot].wait()
        pltpu.sync_copy(bufs[slot], out_hbm.at[pl.ds(rbase + c * CH, CH)])


def kernel(r_idx, r_weight, kv):
    del r_weight
    idx_flat = r_idx.reshape(B)
    kv_flat = kv.reshape(R, CKV)
    out_flat = _sc_gather(idx_flat, kv_flat)
    return out_flat.reshape(N, P2, W2, TOPK, CKV)


# speedup vs baseline: 9.4073x; 1.0080x over previous
import functools

import jax
import jax.numpy as jnp
from jax import lax
from jax.experimental import pallas as pl
from jax.experimental.pallas import tpu as pltpu
from jax.experimental.pallas import tpu_sc as plsc

N, P2, W2, TOPK, CKV = 8, 49, 64, 4, 192
R = N * P2 * W2
B = N * P2 * W2 * TOPK
NC, NS, L = 2, 16, 16
NW = NC * NS
BPW = B // NW
CH = 112
NCH = BPW // CH
VPC = CH // L

_mesh = plsc.VectorSubcoreMesh(core_axis_name="c", subcore_axis_name="s")


@functools.partial(
    pl.kernel,
    mesh=_mesh,
    compiler_params=pltpu.CompilerParams(use_tc_tiling_on_sc=False),
    out_type=jax.ShapeDtypeStruct((B, CKV), jnp.float32),
    scratch_types=[
        pltpu.VMEM((BPW,), jnp.int32),
        pltpu.VMEM((NCH, CH), jnp.int32),
        pltpu.VMEM((4, CH, CKV), jnp.float32),
        [pltpu.SemaphoreType.DMA] * 4,
        [pltpu.SemaphoreType.DMA] * 4,
    ],
)
def _sc_gather(idx_hbm, kv_hbm, out_hbm, idx_v, gidx_v, ring, gsems, wsems):
    wid = lax.axis_index("s") * NC + lax.axis_index("c")
    rbase = wid * BPW

    pltpu.sync_copy(idx_hbm.at[pl.ds(rbase, BPW)], idx_v)

    lane = lax.iota(jnp.int32, 16)

    def ibody(c, _):
        for j in range(VPC):
            off = c * CH + j * L
            rid = rbase + off + lane
            gidx_v[c, pl.ds(j * L, L)] = idx_v[pl.ds(off, L)] + ((rid >> 8) << 6)
        return 0

    lax.fori_loop(0, NCH, ibody, 0)

    def gather(c, s):
        return pltpu.async_copy(kv_hbm.at[gidx_v.at[c]], ring.at[s], gsems[s])

    def write(c, s):
        return pltpu.async_copy(ring.at[s], out_hbm.at[pl.ds(rbase + c * CH, CH)],
                                wsems[s])

    D = 4
    g = [None] * D
    w = [None] * D
    for c in range(min(3, NCH)):
        g[c % D] = gather(c, c % D)
    for c in range(NCH):
        s = c % D
        g[s].wait()
        w[s] = write(c, s)
        nxt = c + 3
        if nxt < NCH:
            p = nxt % D
            if c >= 1:
                w[p].wait()
            g[p] = gather(nxt, p)
    for c in range(max(0, NCH - D), NCH):
        w[c % D].wait()


def kernel(r_idx, r_weight, kv):
    del r_weight
    idx_flat = r_idx.reshape(B)
    kv_flat = kv.reshape(R, CKV)
    out_flat = _sc_gather(idx_flat, kv_flat)
    return out_flat.reshape(N, P2, W2, TOPK, CKV)
